# Initial kernel scaffold; baseline (speedup 1.0000x reference)
import jax
import jax.numpy as jnp
from jax.experimental import pallas as pl


def kernel(x):
    raise NotImplementedError("write your pallas kernel here")


# speedup vs baseline: 1.4403x; 1.4403x over previous
import jax
import jax.numpy as jnp
from jax.experimental import pallas as pl
from jax.experimental.pallas import tpu as pltpu


def _unshuffle_pad_kernel(x_ref, o_ref):
    xv = x_ref[0, 0]
    i = jax.lax.broadcasted_iota(jnp.int32, (256, 256), 0)
    j = jax.lax.broadcasted_iota(jnp.int32, (256, 256), 1)
    D = (i == 2 * (j % 128) + j // 128).astype(jnp.float32)
    Y = []
    for q in range(4):
        Y.append(
            jnp.dot(
                xv[:, q * 256 : (q + 1) * 256],
                D,
                preferred_element_type=jnp.float32,
                precision=jax.lax.Precision.HIGHEST,
            )
        )
    for s1 in range(2):
        for s2 in range(2):
            core = jnp.concatenate(
                [
                    Y[2 * s1][:, s2 * 128 : (s2 + 1) * 128],
                    Y[2 * s1 + 1][:, s2 * 128 : (s2 + 1) * 128],
                ],
                axis=1,
            )
            rows = jnp.concatenate([core[0:1, :], core, core[255:256, :]], axis=0)
            full = jnp.concatenate([rows[:, 0:1], rows, rows[:, 255:256]], axis=1)
            o_ref[0, 0, s1, s2] = full


def kernel(x):
    B, C, H, W = x.shape
    Ho, Wo = H // 2 + 2, W // 2 + 2
    xv = x.reshape(B, C, H // 2, 2 * W)
    out6 = pl.pallas_call(
        _unshuffle_pad_kernel,
        grid=(B, C),
        in_specs=[pl.BlockSpec((1, 1, H // 2, 2 * W), lambda b, c: (b, c, 0, 0))],
        out_specs=pl.BlockSpec(
            (1, 1, 2, 2, Ho, Wo), lambda b, c: (b, c, 0, 0, 0, 0)
        ),
        out_shape=jax.ShapeDtypeStruct((B, C, 2, 2, Ho, Wo), x.dtype),
        compiler_params=pltpu.CompilerParams(
            dimension_semantics=("parallel", "parallel"),
        ),
    )(xv)
    return out6.reshape(B, C * 4, Ho, Wo)


# speedup vs baseline: 1.8328x; 1.2725x over previous
import jax
import jax.numpy as jnp
from jax.experimental import pallas as pl
from jax.experimental.pallas import tpu as pltpu


def _ta(arr, idx):
    return jnp.take_along_axis(arr, idx, axis=1)


def _unshuffle_pad_kernel(x_ref, o_ref):
    x = x_ref[0, 0]
    ii = jax.lax.broadcasted_iota(jnp.int32, (256, 256), 0)
    jj = jax.lax.broadcasted_iota(jnp.int32, (256, 256), 1)
    D = (ii == 2 * (jj % 128) + jj // 128).astype(jnp.bfloat16)
    xh = x.astype(jnp.bfloat16)
    xl = (x - xh.astype(jnp.float32)).astype(jnp.bfloat16)
    y = jnp.concatenate(
        [
            jnp.dot(
                xh[:, h * 256 : (h + 1) * 256],
                D,
                preferred_element_type=jnp.float32,
            )
            + jnp.dot(
                xl[:, h * 256 : (h + 1) * 256],
                D,
                preferred_element_type=jnp.float32,
            )
            for h in range(2)
        ],
        axis=1,
    )
    y4 = y.reshape(32, 2, 8, 512)
    ye = y4[:, 0]
    yo = y4[:, 1]
    yp = jnp.roll(yo, 1, axis=0)
    si = jax.lax.broadcasted_iota(jnp.int32, (32, 8, 512), 1)
    row = jax.lax.broadcasted_iota(jnp.int32, (256, 512), 0)
    for s1 in range(2):
        q = (2 * si - 2 + s1) % 8
        g = jnp.where(
            si == 0,
            _ta(yp, q),
            jnp.where(si <= 4, _ta(ye, q), _ta(yo, q)),
        ).reshape(256, 512)
        g = jnp.where(row == 0, jnp.broadcast_to(y[s1 : s1 + 1, :], (256, 512)), g)
        gt = jnp.broadcast_to(y[510 + s1 : 511 + s1, :], (2, 512))
        z = jnp.concatenate([g, gt], axis=0)
        for s2 in range(2):
            core = jnp.concatenate(
                [
                    z[:, 128 * s2 : 128 * s2 + 128],
                    z[:, 256 + 128 * s2 : 256 + 128 * s2 + 128],
                ],
                axis=1,
            )
            full = jnp.concatenate(
                [core[:, 0:1], core, core[:, 255:256]], axis=1
            )
            o_ref[0, 2 * s1 + s2] = full


def kernel(x):
    B, C, H, W = x.shape
    Ho, Wo = H // 2 + 2, W // 2 + 2
    return pl.pallas_call(
        _unshuffle_pad_kernel,
        grid=(B, C),
        in_specs=[pl.BlockSpec((1, 1, H, W), lambda b, c: (b, c, 0, 0))],
        out_specs=pl.BlockSpec((1, 4, Ho, Wo), lambda b, c: (b, c, 0, 0)),
        out_shape=jax.ShapeDtypeStruct((B, 4 * C, Ho, Wo), x.dtype),
        compiler_params=pltpu.CompilerParams(
            dimension_semantics=("parallel", "parallel"),
        ),
    )(x)


# speedup vs baseline: 2.1747x; 1.1865x over previous
import jax
import jax.numpy as jnp
from jax.experimental import pallas as pl
from jax.experimental.pallas import tpu as pltpu


def _ta(arr, idx):
    return jnp.take_along_axis(arr, idx, axis=1)


_CB = 4


def _unshuffle_pad_kernel(x_ref, o_ref):
    ii = jax.lax.broadcasted_iota(jnp.int32, (256, 256), 0)
    jj = jax.lax.broadcasted_iota(jnp.int32, (256, 256), 1)
    D = (ii == 2 * (jj % 128) + jj // 128).astype(jnp.bfloat16)
    for ci in range(_CB):
        _one_plane(x_ref[0, ci], o_ref.at[0, 4 * ci : 4 * ci + 4], D)


def _one_plane(x, o_ref, D):
    xh = x.astype(jnp.bfloat16)
    xl = (x - xh.astype(jnp.float32)).astype(jnp.bfloat16)
    y = jnp.concatenate(
        [
            jnp.dot(
                xh[:, h * 256 : (h + 1) * 256],
                D,
                preferred_element_type=jnp.float32,
            )
            + jnp.dot(
                xl[:, h * 256 : (h + 1) * 256],
                D,
                preferred_element_type=jnp.float32,
            )
            for h in range(2)
        ],
        axis=1,
    )
    y4 = y.reshape(32, 2, 8, 512)
    ye = y4[:, 0]
    yo = y4[:, 1]
    yp = jnp.roll(yo, 1, axis=0)
    si = jax.lax.broadcasted_iota(jnp.int32, (32, 8, 512), 1)
    row = jax.lax.broadcasted_iota(jnp.int32, (256, 512), 0)
    for s1 in range(2):
        q = (2 * si - 2 + s1) % 8
        g = jnp.where(
            si == 0,
            _ta(yp, q),
            jnp.where(si <= 4, _ta(ye, q), _ta(yo, q)),
        ).reshape(256, 512)
        g = jnp.where(row == 0, jnp.broadcast_to(y[s1 : s1 + 1, :], (256, 512)), g)
        gt = jnp.broadcast_to(y[510 + s1 : 511 + s1, :], (2, 512))
        z = jnp.concatenate([g, gt], axis=0)
        for s2 in range(2):
            core = jnp.concatenate(
                [
                    z[:, 128 * s2 : 128 * s2 + 128],
                    z[:, 256 + 128 * s2 : 256 + 128 * s2 + 128],
                ],
                axis=1,
            )
            full = jnp.concatenate(
                [core[:, 0:1], core, core[:, 255:256]], axis=1
            )
            o_ref[2 * s1 + s2] = full


def kernel(x):
    B, C, H, W = x.shape
    Ho, Wo = H // 2 + 2, W // 2 + 2
    return pl.pallas_call(
        _unshuffle_pad_kernel,
        grid=(B, C // _CB),
        in_specs=[pl.BlockSpec((1, _CB, H, W), lambda b, c: (b, c, 0, 0))],
        out_specs=pl.BlockSpec((1, 4 * _CB, Ho, Wo), lambda b, c: (b, c, 0, 0)),
        out_shape=jax.ShapeDtypeStruct((B, 4 * C, Ho, Wo), x.dtype),
        compiler_params=pltpu.CompilerParams(
            dimension_semantics=("parallel", "parallel"),
        ),
    )(x)


# speedup vs baseline: 2.2158x; 1.0189x over previous
import jax
import jax.numpy as jnp
from jax.experimental import pallas as pl
from jax.experimental.pallas import tpu as pltpu


def _ta(arr, idx):
    return jnp.take_along_axis(arr, idx, axis=1)


_CB = 6


def _unshuffle_pad_kernel(x_ref, o_ref):
    ii = jax.lax.broadcasted_iota(jnp.int32, (256, 256), 0)
    jj = jax.lax.broadcasted_iota(jnp.int32, (256, 256), 1)
    D = (ii == 2 * (jj % 128) + jj // 128).astype(jnp.bfloat16)
    for ci in range(_CB):
        _one_plane(x_ref[0, ci], o_ref.at[0, 4 * ci : 4 * ci + 4], D)


def _one_plane(x, o_ref, D):
    xh = x.astype(jnp.bfloat16)
    xl = (x - xh.astype(jnp.float32)).astype(jnp.bfloat16)
    y = jnp.concatenate(
        [
            jnp.dot(
                xh[:, h * 256 : (h + 1) * 256],
                D,
                preferred_element_type=jnp.float32,
            )
            + jnp.dot(
                xl[:, h * 256 : (h + 1) * 256],
                D,
                preferred_element_type=jnp.float32,
            )
            for h in range(2)
        ],
        axis=1,
    )
    y4 = y.reshape(32, 2, 8, 512)
    ye = y4[:, 0]
    yo = y4[:, 1]
    yp = jnp.roll(yo, 1, axis=0)
    si = jax.lax.broadcasted_iota(jnp.int32, (32, 8, 512), 1)
    row = jax.lax.broadcasted_iota(jnp.int32, (256, 512), 0)
    for s1 in range(2):
        q = (2 * si - 2 + s1) % 8
        g = jnp.where(
            si == 0,
            _ta(yp, q),
            jnp.where(si <= 4, _ta(ye, q), _ta(yo, q)),
        ).reshape(256, 512)
        g = jnp.where(row == 0, jnp.broadcast_to(y[s1 : s1 + 1, :], (256, 512)), g)
        gt = jnp.broadcast_to(y[510 + s1 : 511 + s1, :], (2, 512))
        z = jnp.concatenate([g, gt], axis=0)
        for s2 in range(2):
            core = jnp.concatenate(
                [
                    z[:, 128 * s2 : 128 * s2 + 128],
                    z[:, 256 + 128 * s2 : 256 + 128 * s2 + 128],
                ],
                axis=1,
            )
            full = jnp.concatenate(
                [core[:, 0:1], core, core[:, 255:256]], axis=1
            )
            o_ref[2 * s1 + s2] = full


def kernel(x):
    B, C, H, W = x.shape
    Ho, Wo = H // 2 + 2, W // 2 + 2
    return pl.pallas_call(
        _unshuffle_pad_kernel,
        grid=(B, C // _CB),
        in_specs=[pl.BlockSpec((1, _CB, H, W), lambda b, c: (b, c, 0, 0))],
        out_specs=pl.BlockSpec((1, 4 * _CB, Ho, Wo), lambda b, c: (b, c, 0, 0)),
        out_shape=jax.ShapeDtypeStruct((B, 4 * C, Ho, Wo), x.dtype),
        compiler_params=pltpu.CompilerParams(
            dimension_semantics=("parallel", "parallel"),
        ),
    )(x)
